# Initial kernel scaffold; baseline (speedup 1.0000x reference)
import jax
import jax.numpy as jnp
from jax.experimental import pallas as pl


def kernel(inputs, hidden_state, w_ih0, w_ih, w_hh, b_ih, b_hh, w_q, b_q):
    raise NotImplementedError("write your pallas kernel here")


# speedup vs baseline: 1.0739x; 1.0739x over previous
import jax
import jax.numpy as jnp
from jax import lax
from jax.experimental import pallas as pl
from jax.experimental.pallas import tpu as pltpu


def _drqn_body(x_ref, h0_ref, w0_ref, wih2_ref, whh1_ref, whh2_ref,
               b1_ref, b2_ref, wq_ref, bq_ref,
               out_ref, hout_ref, pre1_ref, h2seq_ref):
    T, Bl, D = x_ref.shape
    f32 = jnp.float32

    x = x_ref[...].reshape(T * Bl, D)
    pre1_ref[...] = jnp.dot(x, w0_ref[...],
                            preferred_element_type=f32) + b1_ref[...]

    whh1 = whh1_ref[...]
    whh2 = whh2_ref[...]
    wih2 = wih2_ref[...]
    b2 = b2_ref[...]

    h1 = jnp.tanh(pre1_ref[pl.ds(0, Bl), :] +
                  jnp.dot(h0_ref[0].astype(jnp.bfloat16), whh1,
                          preferred_element_type=f32))

    def round_fn(r, carry):
        h1, h2 = carry
        h1b = h1.astype(jnp.bfloat16)
        h2b = h2.astype(jnp.bfloat16)
        row = pl.multiple_of(r * Bl, Bl)
        new_h1 = jnp.tanh(pre1_ref[pl.ds(row, Bl), :] +
                          jnp.dot(h1b, whh1, preferred_element_type=f32))
        new_h2 = jnp.tanh(jnp.dot(h1b, wih2, preferred_element_type=f32) +
                          b2 +
                          jnp.dot(h2b, whh2, preferred_element_type=f32))
        prev_row = pl.multiple_of((r - 1) * Bl, Bl)
        h2seq_ref[pl.ds(prev_row, Bl), :] = new_h2.astype(jnp.bfloat16)
        return new_h1, new_h2

    h1, h2 = lax.fori_loop(1, T, round_fn, (h1, h0_ref[1]), unroll=True)

    h1b = h1.astype(jnp.bfloat16)
    h2_last = jnp.tanh(jnp.dot(h1b, wih2, preferred_element_type=f32) + b2 +
                       jnp.dot(h2.astype(jnp.bfloat16), whh2,
                               preferred_element_type=f32))
    last_row = pl.multiple_of((T - 1) * Bl, Bl)
    h2seq_ref[pl.ds(last_row, Bl), :] = h2_last.astype(jnp.bfloat16)
    hout_ref[0] = h1
    hout_ref[1] = h2_last

    q = jnp.dot(h2seq_ref[...], wq_ref[...], preferred_element_type=f32)
    out_ref[...] = (q + bq_ref[...]).reshape(T, Bl, -1)


def kernel(inputs, hidden_state, w_ih0, w_ih, w_hh, b_ih, b_hh, w_q, b_q):
    B, T, D = inputs.shape
    L, _, H = hidden_state.shape
    R = w_q.shape[0]
    cores = 2 if B % 16 == 0 else 1
    Bl = B // cores

    x_tm = jnp.transpose(inputs, (1, 0, 2)).astype(jnp.bfloat16)

    w0_t = w_ih0.T.astype(jnp.bfloat16)
    wih2_t = w_ih[1].T.astype(jnp.bfloat16)
    whh1_t = w_hh[0].T.astype(jnp.bfloat16)
    whh2_t = w_hh[1].T.astype(jnp.bfloat16)
    bias = b_ih + b_hh
    b1 = bias[0].reshape(1, H)
    b2 = bias[1].reshape(1, H)
    wq_t = w_q.T.astype(jnp.bfloat16)
    bq = b_q.reshape(1, R)

    full = lambda shape: pl.BlockSpec(shape, lambda i: (0,) * len(shape))
    bsplit = lambda shape, dim: pl.BlockSpec(
        shape, lambda i, _d=dim: tuple(
            i if d == _d else 0 for d in range(len(shape))))

    out_tm, h_out = pl.pallas_call(
        _drqn_body,
        grid=(cores,),
        in_specs=[
            bsplit((T, Bl, D), 1),
            bsplit((L, Bl, H), 1),
            full((D, H)),
            full((H, H)),
            full((H, H)),
            full((H, H)),
            full((1, H)),
            full((1, H)),
            full((H, R)),
            full((1, R)),
        ],
        out_specs=(
            bsplit((T, Bl, R), 1),
            bsplit((L, Bl, H), 1),
        ),
        out_shape=(
            jax.ShapeDtypeStruct((T, B, R), jnp.float32),
            jax.ShapeDtypeStruct((L, B, H), jnp.float32),
        ),
        scratch_shapes=[
            pltpu.VMEM((T * Bl, H), jnp.float32),
            pltpu.VMEM((T * Bl, H), jnp.bfloat16),
        ],
        compiler_params=pltpu.CompilerParams(
            dimension_semantics=("parallel",)),
    )(x_tm, hidden_state, w0_t, wih2_t, whh1_t, whh2_t, b1, b2, wq_t, bq)

    out = jnp.transpose(out_tm, (1, 0, 2))
    return out, h_out
